# Initial kernel scaffold; baseline (speedup 1.0000x reference)
import jax
import jax.numpy as jnp
from jax.experimental import pallas as pl


def kernel(idx, weight_embedding, main_modes):
    raise NotImplementedError("write your pallas kernel here")


# speedup vs baseline: 1.6272x; 1.6272x over previous
import functools

import jax
import jax.numpy as jnp
from jax import lax
from jax.experimental import pallas as pl
from jax.experimental.pallas import tpu as pltpu
from jax.experimental.pallas import tpu_sc as plsc

_B = 16384
_D = 128
_Z = 512

_NC = 2
_NS = 16
_NW = _NC * _NS
_BPW = _B // _NW
_CH = 128
_NCH = _BPW // _CH


def _make_sc_gather():
    mesh = plsc.VectorSubcoreMesh(core_axis_name="c", subcore_axis_name="s")

    @functools.partial(
        pl.kernel,
        mesh=mesh,
        out_type=jax.ShapeDtypeStruct((_B, _D), jnp.float32),
        scratch_types=[
            pltpu.VMEM((_NCH, _CH), jnp.int32),
            pltpu.VMEM((_BPW, _D), jnp.float32),
            pltpu.SemaphoreType.DMA,
        ],
    )
    def gather_kernel(table_hbm, idx_hbm, out_hbm, idx_v, rows_v, sem):
        wid = lax.axis_index("s") * _NC + lax.axis_index("c")
        pltpu.sync_copy(idx_hbm.at[wid], idx_v)
        cps = [
            pltpu.async_copy(
                table_hbm.at[idx_v.at[j]],
                rows_v.at[pl.ds(j * _CH, _CH)],
                sem,
            )
            for j in range(_NCH)
        ]
        for cp in cps:
            cp.wait()
        pltpu.sync_copy(rows_v, out_hbm.at[pl.ds(wid * _BPW, _BPW)])

    return gather_kernel


_sc_gather = _make_sc_gather()

_BT = 2048


def _tc_body(rows_ref, modes_ref, out_ref):
    x = rows_ref[...]
    m = jnp.max(x, axis=-1, keepdims=True)
    e = jnp.exp(x - m)
    p = e / jnp.sum(e, axis=-1, keepdims=True)
    z = jnp.dot(p, modes_ref[...], preferred_element_type=jnp.float32)
    ss = jnp.maximum(jnp.sum(z * z, axis=-1, keepdims=True), 1e-24)
    out_ref[...] = z * lax.rsqrt(ss)


@jax.jit
def kernel(idx, weight_embedding, main_modes):
    idx32 = idx.astype(jnp.int32).reshape(_NW, _NCH, _CH)
    rows = _sc_gather(weight_embedding, idx32)
    out = pl.pallas_call(
        _tc_body,
        grid=(_B // _BT,),
        in_specs=[
            pl.BlockSpec((_BT, _D), lambda i: (i, 0)),
            pl.BlockSpec((_D, _Z), lambda i: (0, 0)),
        ],
        out_specs=pl.BlockSpec((_BT, _Z), lambda i: (i, 0)),
        out_shape=jax.ShapeDtypeStruct((_B, _Z), jnp.float32),
    )(rows, main_modes)
    return out[:, None, :]


# speedup vs baseline: 2.4278x; 1.4920x over previous
import functools

import jax
import jax.numpy as jnp
from jax import lax
from jax.experimental import pallas as pl
from jax.experimental.pallas import tpu as pltpu
from jax.experimental.pallas import tpu_sc as plsc

_B = 16384
_D = 128
_Z = 512

_NC = 2
_NS = 16
_NW = _NC * _NS
_BPW = _B // _NW
_CH = 128
_NCH = _BPW // _CH


def _make_sc_gather():
    mesh = plsc.VectorSubcoreMesh(core_axis_name="c", subcore_axis_name="s")

    @functools.partial(
        pl.kernel,
        mesh=mesh,
        out_type=jax.ShapeDtypeStruct((_B, _D), jnp.float32),
        scratch_types=[
            pltpu.VMEM((_NCH, _CH), jnp.int32),
            pltpu.VMEM((_BPW, _D), jnp.float32),
            pltpu.SemaphoreType.DMA,
        ],
    )
    def gather_kernel(table_hbm, idx_hbm, out_hbm, idx_v, rows_v, sem):
        wid = lax.axis_index("s") * _NC + lax.axis_index("c")
        pltpu.sync_copy(idx_hbm.at[wid], idx_v)
        cps = [
            pltpu.async_copy(
                table_hbm.at[idx_v.at[j]],
                rows_v.at[pl.ds(j * _CH, _CH)],
                sem,
            )
            for j in range(_NCH)
        ]
        for cp in cps:
            cp.wait()
        pltpu.sync_copy(rows_v, out_hbm.at[pl.ds(wid * _BPW, _BPW)])

    return gather_kernel


_sc_gather = _make_sc_gather()

_BT = 2048


def _tc_body(rows_ref, modes_ref, out_ref):
    x = rows_ref[...]
    m = jnp.max(x, axis=-1, keepdims=True)
    e = jnp.exp(x - m)
    p = e / jnp.sum(e, axis=-1, keepdims=True)
    z = jnp.dot(p, modes_ref[...], preferred_element_type=jnp.float32)
    ss = jnp.maximum(jnp.sum(z * z, axis=-1, keepdims=True), 1e-24)
    out_ref[...] = (z * lax.rsqrt(ss))[:, None, :]


@jax.jit
def kernel(idx, weight_embedding, main_modes):
    idx32 = idx.astype(jnp.int32).reshape(_NW, _NCH, _CH)
    rows = _sc_gather(weight_embedding, idx32)
    out = pl.pallas_call(
        _tc_body,
        grid=(_B // _BT,),
        in_specs=[
            pl.BlockSpec((_BT, _D), lambda i: (i, 0)),
            pl.BlockSpec((_D, _Z), lambda i: (0, 0)),
        ],
        out_specs=pl.BlockSpec((_BT, 1, _Z), lambda i: (i, 0, 0)),
        out_shape=jax.ShapeDtypeStruct((_B, 1, _Z), jnp.float32),
    )(rows, main_modes)
    return out
